# Initial kernel scaffold; baseline (speedup 1.0000x reference)
import functools

import jax
import jax.numpy as jnp
from jax import lax
from jax.experimental import pallas as pl
from jax.experimental.pallas import tpu as pltpu
from jax.experimental.pallas import tpu_sc as plsc

N = 10000
E = 160000
D = 256
DH = 128
NC = 2
NS = 16
LANES = 16

CH = 128
E_PAD = 163840
EROWS = E_PAD // CH
ROWS_PER_TILE = EROWS // NS

BM = 1000


def _mm_body(x_r, w_r, b_r, o_r):
    h = jnp.dot(x_r[...], w_r[...], preferred_element_type=jnp.float32)
    h = h + b_r[...]
    o_r[0] = h[:, :DH]
    o_r[1] = h[:, DH:]


def _linear_split(x, W, b2):
    return pl.pallas_call(
        _mm_body,
        grid=(N // BM,),
        in_specs=[
            pl.BlockSpec((BM, D), lambda i: (i, 0)),
            pl.BlockSpec((D, D), lambda i: (0, 0)),
            pl.BlockSpec((1, D), lambda i: (0, 0)),
        ],
        out_specs=pl.BlockSpec((NC, BM, DH), lambda i: (0, i, 0)),
        out_shape=jax.ShapeDtypeStruct((NC, N, DH), jnp.float32),
    )(x, W, b2)


def _sc_body(h_hbm, src_hbm, dst_hbm, w_hbm, out_hbm,
             rows_v, idx_v, dst_v, w_v, acc_sh, sem):
    c = lax.axis_index("c")
    s = lax.axis_index("s")
    base_row = s * ROWS_PER_TILE

    pltpu.sync_copy(src_hbm.at[pl.ds(base_row, ROWS_PER_TILE)], idx_v)
    pltpu.sync_copy(dst_hbm.at[pl.ds(base_row, ROWS_PER_TILE)], dst_v)
    pltpu.sync_copy(w_hbm.at[pl.ds(base_row, ROWS_PER_TILE)], w_v)

    off = jnp.full((LANES,), c * N, jnp.int32)

    @pl.loop(0, ROWS_PER_TILE)
    def _(k):
        for j in range(CH // LANES):
            slc = (k, pl.ds(j * LANES, LANES))
            idx_v.at[slc][...] = idx_v.at[slc][...] + off

    zero = jnp.zeros((LANES,), jnp.float32)

    @pl.loop(0, CH)
    def _(e):
        for r in range(DH // LANES):
            rows_v.at[e, pl.ds(r * LANES, LANES)][...] = zero

    rows_out = N // NS
    out_base = s * rows_out
    chunk_sizes = []
    n_left = rows_out
    while n_left > 0:
        chunk_sizes.append(min(CH, n_left))
        n_left -= chunk_sizes[-1]
    pos = 0
    for n in chunk_sizes:
        pltpu.sync_copy(rows_v.at[pl.ds(0, n)],
                        acc_sh.at[pl.ds(out_base + pos, n)])
        pos += n

    plsc.subcore_barrier()

    @pl.loop(0, ROWS_PER_TILE)
    def _(k):
        pltpu.async_copy(h_hbm.at[idx_v.at[k]], rows_v, sem).wait()

        @pl.loop(0, CH)
        def _(e):
            kf = jnp.full((LANES,), k, jnp.int32)
            ef = jnp.full((LANES,), e, jnp.int32)
            wv = plsc.load_gather(w_v, [kf, ef])
            for r in range(DH // LANES):
                slc = (e, pl.ds(r * LANES, LANES))
                rows_v.at[slc][...] = rows_v.at[slc][...] * wv

        pltpu.async_copy(rows_v, acc_sh.at[dst_v.at[k]], sem, add=True).wait()

    plsc.subcore_barrier()

    pos = 0
    for n in chunk_sizes:
        pltpu.sync_copy(acc_sh.at[pl.ds(out_base + pos, n)],
                        rows_v.at[pl.ds(0, n)])

        @pl.loop(0, n)
        def _(e):
            for r in range(DH // LANES):
                slc = (e, pl.ds(r * LANES, LANES))
                rows_v.at[slc][...] = jnp.maximum(rows_v.at[slc][...], 0.0)

        pltpu.sync_copy(rows_v.at[pl.ds(0, n)],
                        out_hbm.at[c, pl.ds(out_base + pos, n)])
        pos += n


@functools.partial(
    pl.kernel,
    out_type=jax.ShapeDtypeStruct((NC, N, DH), jnp.float32),
    mesh=plsc.VectorSubcoreMesh(core_axis_name="c", subcore_axis_name="s"),
    scratch_types=[
        pltpu.VMEM((CH, DH), jnp.float32),
        pltpu.VMEM((ROWS_PER_TILE, CH), jnp.int32),
        pltpu.VMEM((ROWS_PER_TILE, CH), jnp.int32),
        pltpu.VMEM((ROWS_PER_TILE, CH), jnp.float32),
        pltpu.VMEM_SHARED((N, DH), jnp.float32),
        pltpu.SemaphoreType.DMA,
    ],
)
def _sc_message_passing(h_hbm, src_hbm, dst_hbm, w_hbm, out_hbm,
                        rows_v, idx_v, dst_v, w_v, acc_sh, sem):
    _sc_body(h_hbm, src_hbm, dst_hbm, w_hbm, out_hbm,
             rows_v, idx_v, dst_v, w_v, acc_sh, sem)


@jax.jit
def kernel(x, edge_index, edge_weight, W, b):
    h2 = _linear_split(x, W, b.reshape(1, D))
    h_flat = h2.reshape(NC * N, DH)

    src = edge_index[0].astype(jnp.int32)
    dst = edge_index[1].astype(jnp.int32)
    pad = E_PAD - E
    src_p = jnp.concatenate([src, jnp.zeros((pad,), jnp.int32)]).reshape(EROWS, CH)
    dst_p = jnp.concatenate([dst, jnp.zeros((pad,), jnp.int32)]).reshape(EROWS, CH)
    w_p = jnp.concatenate(
        [edge_weight.astype(jnp.float32), jnp.zeros((pad,), jnp.float32)]
    ).reshape(EROWS, CH)

    out2 = _sc_message_passing(h_flat, src_p, dst_p, w_p)
    return out2.transpose(1, 0, 2).reshape(N, D)


# speedup vs baseline: 2.4980x; 2.4980x over previous
import dataclasses
import functools

import jax
import jax.numpy as jnp
from jax import lax
from jax.experimental import pallas as pl
from jax.experimental.pallas import tpu as pltpu
from jax.experimental.pallas import tpu_sc as plsc

N = 10000
E = 160000
D = 256
DH = 128
NC = 2
NS = 16
LANES = 16

CH = 128
E_PAD = 163840
EROWS = E_PAD // CH
ROWS_PER_TILE = EROWS // NS

BM = 1000


def _mm_body(x_r, w_r, b_r, o_r):
    h = jnp.dot(x_r[...], w_r[...], preferred_element_type=jnp.float32)
    h = h + b_r[...]
    o_r[0] = h[:, :DH]
    o_r[1] = h[:, DH:]


def _linear_split(x, W, b2):
    return pl.pallas_call(
        _mm_body,
        grid=(N // BM,),
        in_specs=[
            pl.BlockSpec((BM, D), lambda i: (i, 0)),
            pl.BlockSpec((D, D), lambda i: (0, 0)),
            pl.BlockSpec((1, D), lambda i: (0, 0)),
        ],
        out_specs=pl.BlockSpec((NC, BM, DH), lambda i: (0, i, 0)),
        out_shape=jax.ShapeDtypeStruct((NC, N, DH), jnp.float32),
    )(x, W, b2)


def _sc_body(h_hbm, src_hbm, dst_hbm, w_hbm, out_hbm,
             rows_v, idx_v, dst_v, w_v, acc_sh, sem):
    c = lax.axis_index("c")
    s = lax.axis_index("s")
    base_row = s * ROWS_PER_TILE

    pltpu.sync_copy(src_hbm.at[pl.ds(base_row, ROWS_PER_TILE)], idx_v)
    pltpu.sync_copy(dst_hbm.at[pl.ds(base_row, ROWS_PER_TILE)], dst_v)
    pltpu.sync_copy(w_hbm.at[pl.ds(base_row, ROWS_PER_TILE)], w_v)

    off = jnp.full((LANES,), c * N, jnp.int32)

    @pl.loop(0, ROWS_PER_TILE)
    def _(k):
        for j in range(CH // LANES):
            slc = (k, pl.ds(j * LANES, LANES))
            idx_v.at[slc][...] = idx_v.at[slc][...] + off

    zero = jnp.zeros((LANES,), jnp.float32)

    @pl.loop(0, CH)
    def _(e):
        for r in range(DH // LANES):
            rows_v.at[e, pl.ds(r * LANES, LANES)][...] = zero

    n_full = N // CH
    n_tail = N - n_full * CH

    def _for_each_owned_chunk(fn):
        for j in range((n_full + NS - 1) // NS):
            i = s + NS * j

            @pl.when(i < n_full)
            def _():
                fn(i * CH, CH)
        if n_tail:
            @pl.when(s == 0)
            def _():
                fn(n_full * CH, n_tail)

    _for_each_owned_chunk(
        lambda r0, n: pltpu.sync_copy(rows_v.at[pl.ds(0, n)],
                                      acc_sh.at[pl.ds(r0, n)]))

    plsc.subcore_barrier()

    @pl.loop(0, ROWS_PER_TILE)
    def _(k):
        pltpu.async_copy(h_hbm.at[idx_v.at[k]], rows_v, sem).wait()

        @pl.loop(0, CH)
        def _(e):
            kf = jnp.full((LANES,), k, jnp.int32)
            ef = jnp.full((LANES,), e, jnp.int32)
            wv = plsc.load_gather(w_v, [kf, ef])
            for r in range(DH // LANES):
                slc = (e, pl.ds(r * LANES, LANES))
                rows_v.at[slc][...] = rows_v.at[slc][...] * wv

        pltpu.async_copy(rows_v, acc_sh.at[dst_v.at[k]], sem, add=True).wait()

    plsc.subcore_barrier()

    def _relu_out(r0, n):
        pltpu.sync_copy(acc_sh.at[pl.ds(r0, n)], rows_v.at[pl.ds(0, n)])

        @pl.loop(0, n)
        def _(e):
            for r in range(DH // LANES):
                slc = (e, pl.ds(r * LANES, LANES))
                rows_v.at[slc][...] = jnp.maximum(rows_v.at[slc][...], 0.0)

        pltpu.sync_copy(rows_v.at[pl.ds(0, n)],
                        out_hbm.at[c, pl.ds(r0, n)])

    _for_each_owned_chunk(_relu_out)


@functools.lru_cache(maxsize=1)
def _sc_message_passing():
    cp = pltpu.CompilerParams()
    if "needs_layout_passes" in pltpu.CompilerParams.__dataclass_fields__:
        cp = dataclasses.replace(cp, needs_layout_passes=False)
    return pl.kernel(
        _sc_body,
        compiler_params=cp,
        out_type=jax.ShapeDtypeStruct((NC, N, DH), jnp.float32),
        mesh=plsc.VectorSubcoreMesh(core_axis_name="c", subcore_axis_name="s",
                                    num_cores=NC, num_subcores=NS),
        scratch_types=[
            pltpu.VMEM((CH, DH), jnp.float32),
            pltpu.VMEM((ROWS_PER_TILE, CH), jnp.int32),
            pltpu.VMEM((ROWS_PER_TILE, CH), jnp.int32),
            pltpu.VMEM((ROWS_PER_TILE, CH), jnp.float32),
            pltpu.VMEM_SHARED((N, DH), jnp.float32),
            pltpu.SemaphoreType.DMA,
        ],
    )


@jax.jit
def kernel(x, edge_index, edge_weight, W, b):
    h2 = _linear_split(x, W, b.reshape(1, D))
    h_flat = h2.reshape(NC * N, DH)

    src = edge_index[0].astype(jnp.int32)
    dst = edge_index[1].astype(jnp.int32)
    pad = E_PAD - E
    src_p = jnp.concatenate([src, jnp.zeros((pad,), jnp.int32)]).reshape(EROWS, CH)
    dst_p = jnp.concatenate([dst, jnp.zeros((pad,), jnp.int32)]).reshape(EROWS, CH)
    w_p = jnp.concatenate(
        [edge_weight.astype(jnp.float32), jnp.zeros((pad,), jnp.float32)]
    ).reshape(EROWS, CH)

    out2 = _sc_message_passing()(h_flat, src_p, dst_p, w_p)
    return out2.transpose(1, 0, 2).reshape(N, D)


# speedup vs baseline: 3.1629x; 1.2662x over previous
import dataclasses
import functools

import jax
import jax.numpy as jnp
from jax import lax
from jax.experimental import pallas as pl
from jax.experimental.pallas import tpu as pltpu
from jax.experimental.pallas import tpu_sc as plsc

N = 10000
E = 160000
D = 256
DH = 128
NC = 2
NS = 16
LANES = 16

CH = 128
E_PAD = 163840
EROWS = E_PAD // CH
ROWS_PER_TILE = EROWS // NS
STAGE = 40
PHASES = ROWS_PER_TILE // STAGE

BM = 1000


def _mm_body(x_r, w_r, b_r, o_r):
    h = jnp.dot(x_r[...], w_r[...], preferred_element_type=jnp.float32)
    h = h + b_r[...]
    o_r[0] = h[:, :DH]
    o_r[1] = h[:, DH:]


def _linear_split(x, W, b2):
    return pl.pallas_call(
        _mm_body,
        grid=(N // BM,),
        in_specs=[
            pl.BlockSpec((BM, D), lambda i: (i, 0)),
            pl.BlockSpec((D, D), lambda i: (0, 0)),
            pl.BlockSpec((1, D), lambda i: (0, 0)),
        ],
        out_specs=pl.BlockSpec((NC, BM, DH), lambda i: (0, i, 0)),
        out_shape=jax.ShapeDtypeStruct((NC, N, DH), jnp.float32),
    )(x, W, b2)


def _sc_body(h_hbm, src_hbm, dst_hbm, w_hbm, out_hbm,
             rows_v, rows1_v, idx_v, dst_v, w_v, acc_sh, g0, g1, s0, s1):
    c = lax.axis_index("c")
    s = lax.axis_index("s")
    base_row = s * ROWS_PER_TILE

    off = jnp.full((LANES,), c * N, jnp.int32)

    zero = jnp.zeros((LANES,), jnp.float32)

    @pl.loop(0, CH)
    def _(e):
        for r in range(DH // LANES):
            rows_v.at[e, pl.ds(r * LANES, LANES)][...] = zero

    n_full = N // CH
    n_tail = N - n_full * CH

    def _for_each_owned_chunk(fn):
        for j in range((n_full + NS - 1) // NS):
            i = s + NS * j

            @pl.when(i < n_full)
            def _():
                fn(i * CH, CH)
        if n_tail:
            @pl.when(s == 0)
            def _():
                fn(n_full * CH, n_tail)

    _for_each_owned_chunk(
        lambda r0, n: pltpu.sync_copy(rows_v.at[pl.ds(0, n)],
                                      acc_sh.at[pl.ds(r0, n)]))

    plsc.subcore_barrier()

    def _start_gather(k, buf, sem):
        pltpu.async_copy(h_hbm.at[idx_v.at[k]], buf, sem)

    def _start_scatter(k, buf, sem):
        pltpu.async_copy(buf, acc_sh.at[dst_v.at[k]], sem, add=True)

    def _drain(sem):
        pltpu.make_async_copy(h_hbm.at[pl.ds(0, CH)], rows_v, sem).wait()

    def _scale(k, buf):
        kf = jnp.full((LANES,), k, jnp.int32)

        @plsc.parallel_loop(0, CH, unroll=4)
        def _(e):
            ef = jnp.full((LANES,), e, jnp.int32)
            wv = plsc.load_gather(w_v, [kf, ef])
            for r in range(DH // LANES):
                slc = (e, pl.ds(r * LANES, LANES))
                buf.at[slc][...] = buf.at[slc][...] * wv

    for p in range(PHASES):
        stage_base = base_row + p * STAGE
        pltpu.sync_copy(src_hbm.at[pl.ds(stage_base, STAGE)], idx_v)
        pltpu.sync_copy(dst_hbm.at[pl.ds(stage_base, STAGE)], dst_v)
        pltpu.sync_copy(w_hbm.at[pl.ds(stage_base, STAGE)], w_v)

        @pl.loop(0, STAGE)
        def _(k):
            for j in range(CH // LANES):
                slc = (k, pl.ds(j * LANES, LANES))
                idx_v.at[slc][...] = idx_v.at[slc][...] + off

        _start_gather(0, rows_v, g0)

        @pl.loop(0, STAGE, step=2)
        def _(k):
            _drain(g0)

            @pl.when(k > 0)
            def _():
                _drain(s1)

            _start_gather(k + 1, rows1_v, g1)
            _scale(k, rows_v)
            _start_scatter(k, rows_v, s0)
            _drain(g1)
            _scale(k + 1, rows1_v)
            _drain(s0)

            @pl.when(k + 2 < STAGE)
            def _():
                _start_gather(k + 2, rows_v, g0)

            _start_scatter(k + 1, rows1_v, s1)

        _drain(s1)

    plsc.subcore_barrier()

    def _relu_out(r0, n):
        pltpu.sync_copy(acc_sh.at[pl.ds(r0, n)], rows_v.at[pl.ds(0, n)])

        @pl.loop(0, n)
        def _(e):
            for r in range(DH // LANES):
                slc = (e, pl.ds(r * LANES, LANES))
                rows_v.at[slc][...] = jnp.maximum(rows_v.at[slc][...], 0.0)

        pltpu.sync_copy(rows_v.at[pl.ds(0, n)],
                        out_hbm.at[c, pl.ds(r0, n)])

    _for_each_owned_chunk(_relu_out)


@functools.lru_cache(maxsize=1)
def _sc_message_passing():
    cp = pltpu.CompilerParams()
    if "needs_layout_passes" in pltpu.CompilerParams.__dataclass_fields__:
        cp = dataclasses.replace(cp, needs_layout_passes=False)
    return pl.kernel(
        _sc_body,
        compiler_params=cp,
        out_type=jax.ShapeDtypeStruct((NC, N, DH), jnp.float32),
        mesh=plsc.VectorSubcoreMesh(core_axis_name="c", subcore_axis_name="s",
                                    num_cores=NC, num_subcores=NS),
        scratch_types=[
            pltpu.VMEM((CH, DH), jnp.float32),
            pltpu.VMEM((CH, DH), jnp.float32),
            pltpu.VMEM((STAGE, CH), jnp.int32),
            pltpu.VMEM((STAGE, CH), jnp.int32),
            pltpu.VMEM((STAGE, CH), jnp.float32),
            pltpu.VMEM_SHARED((N, DH), jnp.float32),
            pltpu.SemaphoreType.DMA,
            pltpu.SemaphoreType.DMA,
            pltpu.SemaphoreType.DMA,
            pltpu.SemaphoreType.DMA,
        ],
    )


@jax.jit
def kernel(x, edge_index, edge_weight, W, b):
    h2 = _linear_split(x, W, b.reshape(1, D))
    h_flat = h2.reshape(NC * N, DH)

    src = edge_index[0].astype(jnp.int32)
    dst = edge_index[1].astype(jnp.int32)
    pad = E_PAD - E
    src_p = jnp.concatenate([src, jnp.zeros((pad,), jnp.int32)]).reshape(EROWS, CH)
    dst_p = jnp.concatenate([dst, jnp.zeros((pad,), jnp.int32)]).reshape(EROWS, CH)
    w_p = jnp.concatenate(
        [edge_weight.astype(jnp.float32), jnp.zeros((pad,), jnp.float32)]
    ).reshape(EROWS, CH)

    out2 = _sc_message_passing()(h_flat, src_p, dst_p, w_p)
    return out2.transpose(1, 0, 2).reshape(N, D)


# speedup vs baseline: 3.3693x; 1.0652x over previous
import dataclasses
import functools

import jax
import jax.numpy as jnp
from jax import lax
from jax.experimental import pallas as pl
from jax.experimental.pallas import tpu as pltpu
from jax.experimental.pallas import tpu_sc as plsc

N = 10000
E = 160000
D = 256
NC = 2
NS = 16
LANES = 16
HALF = N // NC
QTR = HALF // 2
PASSES = 2

E_PAD = 163840
EROWS = E_PAD // 128
RPT = EROWS // NS
SROWS = 8
SPHASES = RPT // SROWS

CH = 64
CAP = 10368

BM = 1000
PACK_SHIFT = 14


def _mm_body(x_r, w_r, b_r, o_r):
    h = jnp.dot(x_r[...], w_r[...], preferred_element_type=jnp.float32)
    o_r[...] = h + b_r[...]


def _linear(x, W, b2):
    return pl.pallas_call(
        _mm_body,
        grid=(N // BM,),
        in_specs=[
            pl.BlockSpec((BM, D), lambda i: (i, 0)),
            pl.BlockSpec((D, D), lambda i: (0, 0)),
            pl.BlockSpec((1, D), lambda i: (0, 0)),
        ],
        out_specs=pl.BlockSpec((BM, D), lambda i: (i, 0)),
        out_shape=jax.ShapeDtypeStruct((N, D), jnp.float32),
    )(x, W, b2)


def _sc_body(h_hbm, src_hbm, dst_hbm, w_hbm, out_hbm,
             gbuf0, gbuf1, sbuf0, sbuf1, src_st, dst_st, w_st,
             cpk_v, cw_v, cidx0, cidx1, cdst0, cdst1,
             acc_sh, g0, g1, s0, s1):
    c = lax.axis_index("c")
    s = lax.axis_index("s")
    base_row = s * RPT

    zero = jnp.zeros((LANES,), jnp.float32)
    izero = jnp.zeros((LANES,), jnp.int32)
    iota2 = lax.iota(jnp.int32, LANES) * 2

    ACC_ROWS = 2 * QTR
    n_zch = ACC_ROWS // (2 * CH) + 1
    z_tail = ACC_ROWS - (n_zch - 1) * (2 * CH)

    def _for_each_owned_chunk(fn):
        for j in range((n_zch + NS - 1) // NS):
            i = s + NS * j

            @pl.when(i < n_zch - 1)
            def _():
                fn(i * 2 * CH, 2 * CH)

            @pl.when(i == n_zch - 1)
            def _():
                fn(i * 2 * CH, z_tail)

    def _drain(sem):
        pltpu.make_async_copy(h_hbm.at[pl.ds(0, CH)], gbuf0, sem).wait()

    for q in range(PASSES):
        lo = c * HALF + q * QTR
        lo_v = jnp.full((LANES,), lo, jnp.int32)
        hi_v = jnp.full((LANES,), lo + QTR, jnp.int32)

        @pl.loop(0, CAP, step=LANES)
        def _(i):
            cpk_v.at[pl.ds(i, LANES)][...] = izero
            cw_v.at[pl.ds(i, LANES)][...] = zero

        def _scan_phase(p, pos):
            st_base = base_row + p * SROWS
            pltpu.sync_copy(src_hbm.at[pl.ds(st_base, SROWS)], src_st)
            pltpu.sync_copy(dst_hbm.at[pl.ds(st_base, SROWS)], dst_st)
            pltpu.sync_copy(w_hbm.at[pl.ds(st_base, SROWS)], w_st)

            def _row(r, pos):
                for g in range(128 // LANES):
                    j = g * LANES
                    srcv = src_st.at[r, pl.ds(j, LANES)][...]
                    dstv = dst_st.at[r, pl.ds(j, LANES)][...]
                    wv = w_st.at[r, pl.ds(j, LANES)][...]
                    m = (dstv >= lo_v) & (dstv < hi_v)
                    pk = srcv | ((dstv - lo_v) << PACK_SHIFT)
                    plsc.store_compressed(cpk_v.at[pl.ds(pos, LANES)], pk,
                                          mask=m)
                    plsc.store_compressed(cw_v.at[pl.ds(pos, LANES)], wv,
                                          mask=m)
                    pos = pos + jnp.sum(m.astype(jnp.int32))
                return pos

            return lax.fori_loop(0, SROWS, _row, pos)

        pos = jnp.int32(0)
        for p in range(SPHASES):
            pos = _scan_phase(p, pos)

        @pl.loop(0, 2 * CH)
        def _(e):
            for r in range(128 // LANES):
                sbuf0.at[e, pl.ds(r * LANES, LANES)][...] = zero

        _for_each_owned_chunk(
            lambda r0, n: pltpu.sync_copy(sbuf0.at[pl.ds(0, n)],
                                          acc_sh.at[pl.ds(r0, n)]))

        plsc.subcore_barrier()

        nch = jnp.maximum(((pos + 2 * CH - 1) // (2 * CH)) * 2, 2)

        def _unpack(t, cidx, cdst):
            for j in range(CH // LANES):
                pk = cpk_v.at[pl.ds(t * CH + j * LANES, LANES)][...]
                cidx.at[pl.ds(j * LANES, LANES)][...] = (
                    pk & ((1 << PACK_SHIFT) - 1))
                d2 = (pk >> PACK_SHIFT) * 2
                base = iota2 + (j * 2 * LANES)
                plsc.store_scatter(cdst, [base], d2)
                plsc.store_scatter(cdst, [base + 1], d2 + 1)

        def _start_gather(cidx, gbuf, sem):
            pltpu.async_copy(h_hbm.at[cidx], gbuf, sem)

        def _start_scatter(cdst, sbuf, sem):
            pltpu.async_copy(sbuf, acc_sh.at[cdst], sem, add=True)

        def _scale(t, gbuf, sbuf):
            @plsc.parallel_loop(0, CH, unroll=2)
            def _(e):
                ef = jnp.full((LANES,), t * CH + e, jnp.int32)
                wv = plsc.load_gather(cw_v, [ef])
                for r in range(D // LANES):
                    src_slc = (e, pl.ds(r * LANES, LANES))
                    dst_slc = (2 * e + r // 8, pl.ds((r % 8) * LANES, LANES))
                    sbuf.at[dst_slc][...] = gbuf.at[src_slc][...] * wv

        _unpack(0, cidx0, cdst0)
        _start_gather(cidx0, gbuf0, g0)

        @pl.loop(0, nch, step=2)
        def _(t):
            _drain(g0)

            @pl.when(t > 0)
            def _():
                _drain(s1)

            _unpack(t + 1, cidx1, cdst1)
            _start_gather(cidx1, gbuf1, g1)
            _scale(t, gbuf0, sbuf0)
            _start_scatter(cdst0, sbuf0, s0)
            _drain(g1)
            _scale(t + 1, gbuf1, sbuf1)
            _drain(s0)

            @pl.when(t + 2 < nch)
            def _():
                _unpack(t + 2, cidx0, cdst0)
                _start_gather(cidx0, gbuf0, g0)

            _start_scatter(cdst1, sbuf1, s1)

        _drain(s1)

        plsc.subcore_barrier()

        def _relu_out(r0, n):
            pltpu.sync_copy(acc_sh.at[pl.ds(r0, n)], sbuf0.at[pl.ds(0, n)])

            @pl.loop(0, n)
            def _(e):
                for r in range(128 // LANES):
                    slc = (e, pl.ds(r * LANES, LANES))
                    sbuf0.at[slc][...] = jnp.maximum(sbuf0.at[slc][...], 0.0)

            pltpu.sync_copy(sbuf0.at[pl.ds(0, n)],
                            out_hbm.at[pl.ds(2 * lo + r0, n)])

        _for_each_owned_chunk(_relu_out)

        if q + 1 < PASSES:
            plsc.subcore_barrier()


@functools.lru_cache(maxsize=1)
def _sc_message_passing():
    cp = pltpu.CompilerParams()
    if "needs_layout_passes" in pltpu.CompilerParams.__dataclass_fields__:
        cp = dataclasses.replace(cp, needs_layout_passes=False)
    return pl.kernel(
        _sc_body,
        compiler_params=cp,
        out_type=jax.ShapeDtypeStruct((2 * N, 128), jnp.float32),
        mesh=plsc.VectorSubcoreMesh(core_axis_name="c", subcore_axis_name="s",
                                    num_cores=NC, num_subcores=NS),
        scratch_types=[
            pltpu.VMEM((CH, D), jnp.float32),
            pltpu.VMEM((CH, D), jnp.float32),
            pltpu.VMEM((2 * CH, 128), jnp.float32),
            pltpu.VMEM((2 * CH, 128), jnp.float32),
            pltpu.VMEM((SROWS, 128), jnp.int32),
            pltpu.VMEM((SROWS, 128), jnp.int32),
            pltpu.VMEM((SROWS, 128), jnp.float32),
            pltpu.VMEM((CAP,), jnp.int32),
            pltpu.VMEM((CAP,), jnp.float32),
            pltpu.VMEM((CH,), jnp.int32),
            pltpu.VMEM((CH,), jnp.int32),
            pltpu.VMEM((2 * CH,), jnp.int32),
            pltpu.VMEM((2 * CH,), jnp.int32),
            pltpu.VMEM_SHARED((2 * QTR, 128), jnp.float32),
            pltpu.SemaphoreType.DMA,
            pltpu.SemaphoreType.DMA,
            pltpu.SemaphoreType.DMA,
            pltpu.SemaphoreType.DMA,
        ],
    )


@jax.jit
def kernel(x, edge_index, edge_weight, W, b):
    h = _linear(x, W, b.reshape(1, D))

    src = edge_index[0].astype(jnp.int32)
    dst = edge_index[1].astype(jnp.int32)
    pad = E_PAD - E
    src_p = jnp.concatenate([src, jnp.zeros((pad,), jnp.int32)]).reshape(EROWS, 128)
    dst_p = jnp.concatenate(
        [dst, jnp.full((pad,), -1, jnp.int32)]).reshape(EROWS, 128)
    w_p = jnp.concatenate(
        [edge_weight.astype(jnp.float32), jnp.zeros((pad,), jnp.float32)]
    ).reshape(EROWS, 128)

    out_raw = _sc_message_passing()(h, src_p, dst_p, w_p)
    return out_raw.reshape(N, D)
